# Initial kernel scaffold; baseline (speedup 1.0000x reference)
import functools

import jax
import jax.numpy as jnp
from jax import lax
from jax.experimental import pallas as pl
from jax.experimental.pallas import tpu as pltpu
from jax.experimental.pallas import tpu_sc as plsc

NC = 2
NS = 16
NW = NC * NS
CH = 128
DH = 32


def _sc_degree(dstp, zeros16, ones16, nt):
    k = dstp.shape[1]
    rpt = nt // NS
    mesh = plsc.VectorSubcoreMesh(core_axis_name="c", subcore_axis_name="s")

    @functools.partial(
        pl.kernel,
        out_type=jax.ShapeDtypeStruct((NC, nt, 16), jnp.float32),
        mesh=mesh,
        scratch_types=[
            pltpu.VMEM((k, CH), jnp.int32),
            pltpu.VMEM((CH, 16), jnp.float32),
            pltpu.VMEM_SHARED((nt, 16), jnp.float32),
        ],
    )
    def deg_kernel(dstp_hbm, z_hbm, ones_hbm, out_hbm, idx_v, ones_v, acc_sh):
        c = lax.axis_index("c")
        s = lax.axis_index("s")
        wid = c * NS + s
        pltpu.sync_copy(ones_hbm, ones_v)
        pltpu.sync_copy(dstp_hbm.at[wid], idx_v)
        pltpu.sync_copy(z_hbm, acc_sh.at[pl.ds(s * rpt, rpt)])
        plsc.subcore_barrier()

        def step(j, carry):
            pltpu.sync_copy(ones_v, acc_sh.at[idx_v.at[j]], add=True)
            return carry

        lax.fori_loop(0, k, step, 0)
        plsc.subcore_barrier()
        pltpu.sync_copy(acc_sh.at[pl.ds(s * rpt, rpt)],
                        out_hbm.at[c, pl.ds(s * rpt, rpt)])

    return deg_kernel(dstp, zeros16, ones16)


def _sc_scatter(g, srcp, dstp, zeros32, nt):
    k = srcp.shape[1]
    rpt = nt // NS
    mesh = plsc.VectorSubcoreMesh(core_axis_name="c", subcore_axis_name="s")

    @functools.partial(
        pl.kernel,
        out_type=jax.ShapeDtypeStruct((NC, nt, DH), jnp.float32),
        mesh=mesh,
        scratch_types=[
            pltpu.VMEM((k, CH), jnp.int32),
            pltpu.VMEM((k, CH), jnp.int32),
            pltpu.VMEM((CH, DH), jnp.float32),
            pltpu.VMEM_SHARED((nt, DH), jnp.float32),
            pltpu.SemaphoreType.DMA,
        ],
    )
    def scat_kernel(g_hbm, srcp_hbm, dstp_hbm, z_hbm, out_hbm,
                    isrc_v, idst_v, rows_v, acc_sh, sem):
        c = lax.axis_index("c")
        s = lax.axis_index("s")
        wid = c * NS + s
        pltpu.sync_copy(srcp_hbm.at[wid], isrc_v)
        pltpu.sync_copy(dstp_hbm.at[wid], idst_v)
        pltpu.sync_copy(z_hbm, acc_sh.at[pl.ds(s * rpt, rpt)])
        plsc.subcore_barrier()

        def step(j, carry):
            pltpu.async_copy(g_hbm.at[isrc_v.at[j]], rows_v, sem).wait()
            pltpu.sync_copy(rows_v, acc_sh.at[idst_v.at[j]], add=True)
            return carry

        lax.fori_loop(0, k, step, 0)
        plsc.subcore_barrier()
        pltpu.sync_copy(acc_sh.at[pl.ds(s * rpt, rpt)],
                        out_hbm.at[c, pl.ds(s * rpt, rpt)])

    return scat_kernel(g, srcp, dstp, zeros32)


def _dinv_from(d_ref):
    deg = d_ref[0, :, 0] + d_ref[1, :, 0] + 1.0
    return lax.rsqrt(deg)


def _tc_in(xp, W1, degacc, nt, r):
    def body(x_ref, w_ref, d_ref, o_ref):
        dinv = _dinv_from(d_ref)
        h = jnp.dot(x_ref[...], w_ref[...], preferred_element_type=jnp.float32)
        o_ref[...] = h * dinv[:, None]

    return pl.pallas_call(
        body,
        grid=(nt // r,),
        in_specs=[
            pl.BlockSpec((r, xp.shape[1]), lambda i: (i, 0)),
            pl.BlockSpec(W1.shape, lambda i: (0, 0)),
            pl.BlockSpec((NC, r, 16), lambda i: (0, i, 0)),
        ],
        out_specs=pl.BlockSpec((r, DH), lambda i: (i, 0)),
        out_shape=jax.ShapeDtypeStruct((nt, DH), jnp.float32),
    )(xp, W1, degacc)


def _tc_mid(sacc, g1, degacc, b1, W2, nt, r):
    def body(s_ref, g_ref, d_ref, b_ref, w_ref, o_ref):
        dinv = _dinv_from(d_ref)
        stot = s_ref[0] + s_ref[1] + g_ref[...]
        z = jnp.maximum(stot * dinv[:, None] + b_ref[...], 0.0)
        h = jnp.dot(z, w_ref[...], preferred_element_type=jnp.float32)
        o_ref[...] = h * dinv[:, None]

    return pl.pallas_call(
        body,
        grid=(nt // r,),
        in_specs=[
            pl.BlockSpec((NC, r, DH), lambda i: (0, i, 0)),
            pl.BlockSpec((r, DH), lambda i: (i, 0)),
            pl.BlockSpec((NC, r, 16), lambda i: (0, i, 0)),
            pl.BlockSpec(b1.shape, lambda i: (0, 0)),
            pl.BlockSpec(W2.shape, lambda i: (0, 0)),
        ],
        out_specs=pl.BlockSpec((r, DH), lambda i: (i, 0)),
        out_shape=jax.ShapeDtypeStruct((nt, DH), jnp.float32),
    )(sacc, g1, degacc, b1, W2)


def _tc_head(sacc, g2, degacc, b2, Wfc, bfc, nt, r):
    dout = Wfc.shape[1]

    def body(s_ref, g_ref, d_ref, b_ref, w_ref, bf_ref, o_ref):
        dinv = _dinv_from(d_ref)
        stot = s_ref[0] + s_ref[1] + g_ref[...]
        z = jnp.maximum(stot * dinv[:, None] + b_ref[...], 0.0)
        h = jnp.dot(z, w_ref[...], preferred_element_type=jnp.float32)
        o_ref[...] = jax.nn.sigmoid(h + bf_ref[...])

    return pl.pallas_call(
        body,
        grid=(nt // r,),
        in_specs=[
            pl.BlockSpec((NC, r, DH), lambda i: (0, i, 0)),
            pl.BlockSpec((r, DH), lambda i: (i, 0)),
            pl.BlockSpec((NC, r, 16), lambda i: (0, i, 0)),
            pl.BlockSpec(b2.shape, lambda i: (0, 0)),
            pl.BlockSpec(Wfc.shape, lambda i: (0, 0)),
            pl.BlockSpec(bfc.shape, lambda i: (0, 0)),
        ],
        out_specs=pl.BlockSpec((r, dout), lambda i: (i, 0)),
        out_shape=jax.ShapeDtypeStruct((nt, dout), jnp.float32),
    )(sacc, g2, degacc, b2, Wfc, bfc)


def kernel(x, edge_index, W1, b1, W2, b2, Wfc, bfc):
    n, din = x.shape
    e = edge_index.shape[1]
    r = 1024
    nt = -(-(n + 1) // r) * r
    ept = -(-e // (NW * CH)) * CH
    k = ept // CH
    pad_e = NW * ept - e

    fill = jnp.full((pad_e,), n, dtype=edge_index.dtype)
    srcp = jnp.concatenate([edge_index[0], fill]).reshape(NW, k, CH)
    dstp = jnp.concatenate([edge_index[1], fill]).reshape(NW, k, CH)
    xp = jnp.pad(x, ((0, nt - n), (0, 0)))

    zeros16 = jnp.zeros((nt // NS, 16), jnp.float32)
    zeros32 = jnp.zeros((nt // NS, DH), jnp.float32)
    ones16 = jnp.ones((CH, 16), jnp.float32)

    degacc = _sc_degree(dstp, zeros16, ones16, nt)
    g1 = _tc_in(xp, W1, degacc, nt, r)
    s1 = _sc_scatter(g1, srcp, dstp, zeros32, nt)
    g2 = _tc_mid(s1, g1, degacc, b1.reshape(1, DH), W2, nt, r)
    s2 = _sc_scatter(g2, srcp, dstp, zeros32, nt)

    dpad = 8 - Wfc.shape[1]
    wfc_p = jnp.pad(Wfc, ((0, 0), (0, dpad)))
    bfc_p = jnp.pad(bfc, (0, dpad)).reshape(1, 8)
    outp = _tc_head(s2, g2, degacc, b2.reshape(1, DH), wfc_p, bfc_p, nt, r)
    return outp[:n, :Wfc.shape[1]]


# speedup vs baseline: 24.5526x; 24.5526x over previous
import functools

import jax
import jax.numpy as jnp
from jax import lax
from jax.experimental import pallas as pl
from jax.experimental.pallas import tpu as pltpu
from jax.experimental.pallas import tpu_sc as plsc

NC = 2
NS = 16
NW = NC * NS
CH = 128
DH = 32


def _sc_degree(dstp, zeros16, ones16, nt):
    k = dstp.shape[1]
    rpt = nt // NS
    mesh = plsc.VectorSubcoreMesh(core_axis_name="c", subcore_axis_name="s")

    @functools.partial(
        pl.kernel,
        out_type=jax.ShapeDtypeStruct((NC, nt, 16), jnp.float32),
        mesh=mesh,
        scratch_types=[
            pltpu.VMEM((k, CH), jnp.int32),
            pltpu.VMEM((CH, 16), jnp.float32),
            pltpu.VMEM_SHARED((nt, 16), jnp.float32),
        ],
        compiler_params=pltpu.CompilerParams(use_tc_tiling_on_sc=False),
    )
    def deg_kernel(dstp_hbm, z_hbm, ones_hbm, out_hbm, idx_v, ones_v, acc_sh):
        c = lax.axis_index("c")
        s = lax.axis_index("s")
        wid = c * NS + s
        pltpu.sync_copy(ones_hbm, ones_v)
        pltpu.sync_copy(dstp_hbm.at[wid], idx_v)
        pltpu.sync_copy(z_hbm, acc_sh.at[pl.ds(s * rpt, rpt)])
        plsc.subcore_barrier()

        def step(j, carry):
            pltpu.sync_copy(ones_v, acc_sh.at[idx_v.at[j]], add=True)
            return carry

        lax.fori_loop(0, k, step, 0)
        plsc.subcore_barrier()
        pltpu.sync_copy(acc_sh.at[pl.ds(s * rpt, rpt)],
                        out_hbm.at[c, pl.ds(s * rpt, rpt)])

    return deg_kernel(dstp, zeros16, ones16)


def _sc_scatter(g, srcp, dstp, zeros32, nt):
    k = srcp.shape[1]
    rpt = nt // NS
    mesh = plsc.VectorSubcoreMesh(core_axis_name="c", subcore_axis_name="s")

    @functools.partial(
        pl.kernel,
        out_type=jax.ShapeDtypeStruct((NC, nt, DH), jnp.float32),
        mesh=mesh,
        scratch_types=[
            pltpu.VMEM((k, CH), jnp.int32),
            pltpu.VMEM((k, CH), jnp.int32),
            pltpu.VMEM((CH, DH), jnp.float32),
            pltpu.VMEM_SHARED((nt, DH), jnp.float32),
            pltpu.SemaphoreType.DMA,
        ],
        compiler_params=pltpu.CompilerParams(use_tc_tiling_on_sc=False),
    )
    def scat_kernel(g_hbm, srcp_hbm, dstp_hbm, z_hbm, out_hbm,
                    isrc_v, idst_v, rows_v, acc_sh, sem):
        c = lax.axis_index("c")
        s = lax.axis_index("s")
        wid = c * NS + s
        pltpu.sync_copy(srcp_hbm.at[wid], isrc_v)
        pltpu.sync_copy(dstp_hbm.at[wid], idst_v)
        pltpu.sync_copy(z_hbm, acc_sh.at[pl.ds(s * rpt, rpt)])
        plsc.subcore_barrier()

        def step(j, carry):
            pltpu.async_copy(g_hbm.at[isrc_v.at[j]], rows_v, sem).wait()
            pltpu.sync_copy(rows_v, acc_sh.at[idst_v.at[j]], add=True)
            return carry

        lax.fori_loop(0, k, step, 0)
        plsc.subcore_barrier()
        pltpu.sync_copy(acc_sh.at[pl.ds(s * rpt, rpt)],
                        out_hbm.at[c, pl.ds(s * rpt, rpt)])

    return scat_kernel(g, srcp, dstp, zeros32)


def _dinv_from(d_ref):
    deg = d_ref[0, :, 0] + d_ref[1, :, 0] + 1.0
    return lax.rsqrt(deg)


def _tc_in(xp, W1, degacc, nt, r):
    def body(x_ref, w_ref, d_ref, o_ref):
        dinv = _dinv_from(d_ref)
        h = jnp.dot(x_ref[...], w_ref[...], preferred_element_type=jnp.float32)
        o_ref[...] = h * dinv[:, None]

    return pl.pallas_call(
        body,
        grid=(nt // r,),
        in_specs=[
            pl.BlockSpec((r, xp.shape[1]), lambda i: (i, 0)),
            pl.BlockSpec(W1.shape, lambda i: (0, 0)),
            pl.BlockSpec((NC, r, 16), lambda i: (0, i, 0)),
        ],
        out_specs=pl.BlockSpec((r, DH), lambda i: (i, 0)),
        out_shape=jax.ShapeDtypeStruct((nt, DH), jnp.float32),
    )(xp, W1, degacc)


def _tc_mid(sacc, g1, degacc, b1, W2, nt, r):
    def body(s_ref, g_ref, d_ref, b_ref, w_ref, o_ref):
        dinv = _dinv_from(d_ref)
        stot = s_ref[0] + s_ref[1] + g_ref[...]
        z = jnp.maximum(stot * dinv[:, None] + b_ref[...], 0.0)
        h = jnp.dot(z, w_ref[...], preferred_element_type=jnp.float32)
        o_ref[...] = h * dinv[:, None]

    return pl.pallas_call(
        body,
        grid=(nt // r,),
        in_specs=[
            pl.BlockSpec((NC, r, DH), lambda i: (0, i, 0)),
            pl.BlockSpec((r, DH), lambda i: (i, 0)),
            pl.BlockSpec((NC, r, 16), lambda i: (0, i, 0)),
            pl.BlockSpec(b1.shape, lambda i: (0, 0)),
            pl.BlockSpec(W2.shape, lambda i: (0, 0)),
        ],
        out_specs=pl.BlockSpec((r, DH), lambda i: (i, 0)),
        out_shape=jax.ShapeDtypeStruct((nt, DH), jnp.float32),
    )(sacc, g1, degacc, b1, W2)


def _tc_head(sacc, g2, degacc, b2, Wfc, bfc, nt, r):
    dout = Wfc.shape[1]

    def body(s_ref, g_ref, d_ref, b_ref, w_ref, bf_ref, o_ref):
        dinv = _dinv_from(d_ref)
        stot = s_ref[0] + s_ref[1] + g_ref[...]
        z = jnp.maximum(stot * dinv[:, None] + b_ref[...], 0.0)
        h = jnp.dot(z, w_ref[...], preferred_element_type=jnp.float32)
        o_ref[...] = jax.nn.sigmoid(h + bf_ref[...])

    return pl.pallas_call(
        body,
        grid=(nt // r,),
        in_specs=[
            pl.BlockSpec((NC, r, DH), lambda i: (0, i, 0)),
            pl.BlockSpec((r, DH), lambda i: (i, 0)),
            pl.BlockSpec((NC, r, 16), lambda i: (0, i, 0)),
            pl.BlockSpec(b2.shape, lambda i: (0, 0)),
            pl.BlockSpec(Wfc.shape, lambda i: (0, 0)),
            pl.BlockSpec(bfc.shape, lambda i: (0, 0)),
        ],
        out_specs=pl.BlockSpec((r, dout), lambda i: (i, 0)),
        out_shape=jax.ShapeDtypeStruct((nt, dout), jnp.float32),
    )(sacc, g2, degacc, b2, Wfc, bfc)


def kernel(x, edge_index, W1, b1, W2, b2, Wfc, bfc):
    n, din = x.shape
    e = edge_index.shape[1]
    r = 1024
    nt = -(-(n + 1) // r) * r
    ept = -(-e // (NW * CH)) * CH
    k = ept // CH
    pad_e = NW * ept - e

    fill = jnp.full((pad_e,), n, dtype=edge_index.dtype)
    srcp = jnp.concatenate([edge_index[0], fill]).reshape(NW, k, CH)
    dstp = jnp.concatenate([edge_index[1], fill]).reshape(NW, k, CH)
    xp = jnp.pad(x, ((0, nt - n), (0, 0)))

    zeros16 = jnp.zeros((nt // NS, 16), jnp.float32)
    zeros32 = jnp.zeros((nt // NS, DH), jnp.float32)
    ones16 = jnp.ones((CH, 16), jnp.float32)

    degacc = _sc_degree(dstp, zeros16, ones16, nt)
    g1 = _tc_in(xp, W1, degacc, nt, r)
    s1 = _sc_scatter(g1, srcp, dstp, zeros32, nt)
    g2 = _tc_mid(s1, g1, degacc, b1.reshape(1, DH), W2, nt, r)
    s2 = _sc_scatter(g2, srcp, dstp, zeros32, nt)

    dpad = 8 - Wfc.shape[1]
    wfc_p = jnp.pad(Wfc, ((0, 0), (0, dpad)))
    bfc_p = jnp.pad(bfc, (0, dpad)).reshape(1, 8)
    outp = _tc_head(s2, g2, degacc, b2.reshape(1, DH), wfc_p, bfc_p, nt, r)
    return outp[:n, :Wfc.shape[1]]


# speedup vs baseline: 32.6037x; 1.3279x over previous
import functools

import jax
import jax.numpy as jnp
from jax import lax
from jax.experimental import pallas as pl
from jax.experimental.pallas import tpu as pltpu
from jax.experimental.pallas import tpu_sc as plsc

NC = 2
NS = 16
NW = NC * NS
CH = 128
NB = 4
DH = 32


def _sc_degree(dstp, zeros16, ones16, nt):
    k = dstp.shape[1]
    rpt = nt // NS
    mesh = plsc.VectorSubcoreMesh(core_axis_name="c", subcore_axis_name="s")

    @functools.partial(
        pl.kernel,
        out_type=jax.ShapeDtypeStruct((NC, nt, 16), jnp.float32),
        mesh=mesh,
        scratch_types=[
            pltpu.VMEM((k, CH), jnp.int32),
            pltpu.VMEM((CH, 16), jnp.float32),
            pltpu.VMEM_SHARED((nt, 16), jnp.float32),
        ],
        compiler_params=pltpu.CompilerParams(use_tc_tiling_on_sc=False),
    )
    def deg_kernel(dstp_hbm, z_hbm, ones_hbm, out_hbm, idx_v, ones_v, acc_sh):
        c = lax.axis_index("c")
        s = lax.axis_index("s")
        wid = c * NS + s
        pltpu.sync_copy(ones_hbm, ones_v)
        pltpu.sync_copy(dstp_hbm.at[wid], idx_v)
        pltpu.sync_copy(z_hbm, acc_sh.at[pl.ds(s * rpt, rpt)])
        plsc.subcore_barrier()

        def step(j, carry):
            pltpu.sync_copy(ones_v, acc_sh.at[idx_v.at[j]], add=True)
            return carry

        lax.fori_loop(0, k, step, 0)
        plsc.subcore_barrier()
        pltpu.sync_copy(acc_sh.at[pl.ds(s * rpt, rpt)],
                        out_hbm.at[c, pl.ds(s * rpt, rpt)])

    return deg_kernel(dstp, zeros16, ones16)


def _sc_scatter(g, srcp, dstp, zeros32, nt):
    k = srcp.shape[1]
    rpt = nt // NS
    mesh = plsc.VectorSubcoreMesh(core_axis_name="c", subcore_axis_name="s")

    @functools.partial(
        pl.kernel,
        out_type=jax.ShapeDtypeStruct((NC, nt, DH), jnp.float32),
        mesh=mesh,
        scratch_types=[
            pltpu.VMEM((k, CH), jnp.int32),
            pltpu.VMEM((k, CH), jnp.int32),
            pltpu.VMEM((NB, CH, DH), jnp.float32),
            pltpu.VMEM_SHARED((nt, DH), jnp.float32),
            pltpu.SemaphoreType.DMA((NB,)),
        ],
        compiler_params=pltpu.CompilerParams(use_tc_tiling_on_sc=False),
    )
    def scat_kernel(g_hbm, srcp_hbm, dstp_hbm, z_hbm, out_hbm,
                    isrc_v, idst_v, rows_v, acc_sh, sems):
        c = lax.axis_index("c")
        s = lax.axis_index("s")
        wid = c * NS + s
        pltpu.sync_copy(srcp_hbm.at[wid], isrc_v)
        pltpu.sync_copy(dstp_hbm.at[wid], idst_v)
        pltpu.sync_copy(z_hbm, acc_sh.at[pl.ds(s * rpt, rpt)])
        plsc.subcore_barrier()

        for b in range(min(NB, k)):
            pltpu.async_copy(g_hbm.at[isrc_v.at[b]], rows_v.at[b], sems.at[b])

        def step(j, carry):
            b = lax.rem(j, NB)
            pltpu.make_async_copy(g_hbm.at[isrc_v.at[j]], rows_v.at[b],
                                  sems.at[b]).wait()
            pltpu.sync_copy(rows_v.at[b], acc_sh.at[idst_v.at[j]], add=True)
            nxt = j + NB

            @pl.when(nxt < k)
            def _():
                pltpu.async_copy(g_hbm.at[isrc_v.at[nxt]], rows_v.at[b],
                                 sems.at[b])

            return carry

        lax.fori_loop(0, k, step, 0)
        plsc.subcore_barrier()
        pltpu.sync_copy(acc_sh.at[pl.ds(s * rpt, rpt)],
                        out_hbm.at[c, pl.ds(s * rpt, rpt)])

    return scat_kernel(g, srcp, dstp, zeros32)


def _dinv_from(d_ref):
    deg = d_ref[0, :, 0] + d_ref[1, :, 0] + 1.0
    return lax.rsqrt(deg)


def _tc_in(xp, W1, degacc, nt, r):
    def body(x_ref, w_ref, d_ref, o_ref):
        dinv = _dinv_from(d_ref)
        h = jnp.dot(x_ref[...], w_ref[...], preferred_element_type=jnp.float32)
        o_ref[...] = h * dinv[:, None]

    return pl.pallas_call(
        body,
        grid=(nt // r,),
        in_specs=[
            pl.BlockSpec((r, xp.shape[1]), lambda i: (i, 0)),
            pl.BlockSpec(W1.shape, lambda i: (0, 0)),
            pl.BlockSpec((NC, r, 16), lambda i: (0, i, 0)),
        ],
        out_specs=pl.BlockSpec((r, DH), lambda i: (i, 0)),
        out_shape=jax.ShapeDtypeStruct((nt, DH), jnp.float32),
    )(xp, W1, degacc)


def _tc_mid(sacc, g1, degacc, b1, W2, nt, r):
    def body(s_ref, g_ref, d_ref, b_ref, w_ref, o_ref):
        dinv = _dinv_from(d_ref)
        stot = s_ref[0] + s_ref[1] + g_ref[...]
        z = jnp.maximum(stot * dinv[:, None] + b_ref[...], 0.0)
        h = jnp.dot(z, w_ref[...], preferred_element_type=jnp.float32)
        o_ref[...] = h * dinv[:, None]

    return pl.pallas_call(
        body,
        grid=(nt // r,),
        in_specs=[
            pl.BlockSpec((NC, r, DH), lambda i: (0, i, 0)),
            pl.BlockSpec((r, DH), lambda i: (i, 0)),
            pl.BlockSpec((NC, r, 16), lambda i: (0, i, 0)),
            pl.BlockSpec(b1.shape, lambda i: (0, 0)),
            pl.BlockSpec(W2.shape, lambda i: (0, 0)),
        ],
        out_specs=pl.BlockSpec((r, DH), lambda i: (i, 0)),
        out_shape=jax.ShapeDtypeStruct((nt, DH), jnp.float32),
    )(sacc, g1, degacc, b1, W2)


def _tc_head(sacc, g2, degacc, b2, Wfc, bfc, nt, r):
    dout = Wfc.shape[1]

    def body(s_ref, g_ref, d_ref, b_ref, w_ref, bf_ref, o_ref):
        dinv = _dinv_from(d_ref)
        stot = s_ref[0] + s_ref[1] + g_ref[...]
        z = jnp.maximum(stot * dinv[:, None] + b_ref[...], 0.0)
        h = jnp.dot(z, w_ref[...], preferred_element_type=jnp.float32)
        o_ref[...] = jax.nn.sigmoid(h + bf_ref[...])

    return pl.pallas_call(
        body,
        grid=(nt // r,),
        in_specs=[
            pl.BlockSpec((NC, r, DH), lambda i: (0, i, 0)),
            pl.BlockSpec((r, DH), lambda i: (i, 0)),
            pl.BlockSpec((NC, r, 16), lambda i: (0, i, 0)),
            pl.BlockSpec(b2.shape, lambda i: (0, 0)),
            pl.BlockSpec(Wfc.shape, lambda i: (0, 0)),
            pl.BlockSpec(bfc.shape, lambda i: (0, 0)),
        ],
        out_specs=pl.BlockSpec((r, dout), lambda i: (i, 0)),
        out_shape=jax.ShapeDtypeStruct((nt, dout), jnp.float32),
    )(sacc, g2, degacc, b2, Wfc, bfc)


def kernel(x, edge_index, W1, b1, W2, b2, Wfc, bfc):
    n, din = x.shape
    e = edge_index.shape[1]
    r = 1024
    nt = -(-(n + 1) // r) * r
    ept = -(-e // (NW * CH)) * CH
    k = ept // CH
    pad_e = NW * ept - e

    fill = jnp.full((pad_e,), n, dtype=edge_index.dtype)
    srcp = jnp.concatenate([edge_index[0], fill]).reshape(NW, k, CH)
    dstp = jnp.concatenate([edge_index[1], fill]).reshape(NW, k, CH)
    xp = jnp.pad(x, ((0, nt - n), (0, 0)))

    zeros16 = jnp.zeros((nt // NS, 16), jnp.float32)
    zeros32 = jnp.zeros((nt // NS, DH), jnp.float32)
    ones16 = jnp.ones((CH, 16), jnp.float32)

    degacc = _sc_degree(dstp, zeros16, ones16, nt)
    g1 = _tc_in(xp, W1, degacc, nt, r)
    s1 = _sc_scatter(g1, srcp, dstp, zeros32, nt)
    g2 = _tc_mid(s1, g1, degacc, b1.reshape(1, DH), W2, nt, r)
    s2 = _sc_scatter(g2, srcp, dstp, zeros32, nt)

    dpad = 8 - Wfc.shape[1]
    wfc_p = jnp.pad(Wfc, ((0, 0), (0, dpad)))
    bfc_p = jnp.pad(bfc, (0, dpad)).reshape(1, 8)
    outp = _tc_head(s2, g2, degacc, b2.reshape(1, DH), wfc_p, bfc_p, nt, r)
    return outp[:n, :Wfc.shape[1]]


# speedup vs baseline: 48.4440x; 1.4858x over previous
import functools

import jax
import jax.numpy as jnp
from jax import lax
from jax.experimental import pallas as pl
from jax.experimental.pallas import tpu as pltpu
from jax.experimental.pallas import tpu_sc as plsc

NC = 2
NS = 16
NW = NC * NS
CH = 128
NB = 4
DH = 32


def _sc_degree(dstp, zeros16, ones16, nt):
    k = dstp.shape[1]
    rpt = nt // NS
    mesh = plsc.VectorSubcoreMesh(core_axis_name="c", subcore_axis_name="s")

    @functools.partial(
        pl.kernel,
        out_type=jax.ShapeDtypeStruct((NC, nt, 16), jnp.float32),
        mesh=mesh,
        scratch_types=[
            pltpu.VMEM((k, CH), jnp.int32),
            pltpu.VMEM((CH, 16), jnp.float32),
            pltpu.VMEM_SHARED((nt, 16), jnp.float32),
        ],
        compiler_params=pltpu.CompilerParams(use_tc_tiling_on_sc=False),
    )
    def deg_kernel(dstp_hbm, z_hbm, ones_hbm, out_hbm, idx_v, ones_v, acc_sh):
        c = lax.axis_index("c")
        s = lax.axis_index("s")
        wid = c * NS + s
        pltpu.sync_copy(ones_hbm, ones_v)
        pltpu.sync_copy(dstp_hbm.at[wid], idx_v)
        pltpu.sync_copy(z_hbm, acc_sh.at[pl.ds(s * rpt, rpt)])
        plsc.subcore_barrier()

        def step(j, carry):
            pltpu.sync_copy(ones_v, acc_sh.at[idx_v.at[j]], add=True)
            return carry

        lax.fori_loop(0, k, step, 0)
        plsc.subcore_barrier()
        pltpu.sync_copy(acc_sh.at[pl.ds(s * rpt, rpt)],
                        out_hbm.at[c, pl.ds(s * rpt, rpt)])

    return deg_kernel(dstp, zeros16, ones16)


def _sc_scatter(g, srcp, dstp, zeros32, nt):
    k = srcp.shape[1]
    rpt = nt // NS
    mesh = plsc.VectorSubcoreMesh(core_axis_name="c", subcore_axis_name="s")

    @functools.partial(
        pl.kernel,
        out_type=jax.ShapeDtypeStruct((NC, nt, DH), jnp.float32),
        mesh=mesh,
        scratch_types=[
            pltpu.VMEM((k, CH), jnp.int32),
            pltpu.VMEM((k, CH), jnp.int32),
            pltpu.VMEM((NB, CH, DH), jnp.float32),
            pltpu.VMEM_SHARED((nt, DH), jnp.float32),
            pltpu.SemaphoreType.DMA((NB,)),
        ],
        compiler_params=pltpu.CompilerParams(use_tc_tiling_on_sc=False),
    )
    def scat_kernel(g_hbm, srcp_hbm, dstp_hbm, z_hbm, out_hbm,
                    isrc_v, idst_v, rows_v, acc_sh, sems):
        c = lax.axis_index("c")
        s = lax.axis_index("s")
        wid = c * NS + s
        pltpu.sync_copy(srcp_hbm.at[wid], isrc_v)
        pltpu.sync_copy(dstp_hbm.at[wid], idst_v)
        pltpu.sync_copy(z_hbm, acc_sh.at[pl.ds(s * rpt, rpt)])
        plsc.subcore_barrier()

        for b in range(min(NB, k)):
            pltpu.async_copy(g_hbm.at[isrc_v.at[b]], rows_v.at[b], sems.at[b])

        def step(j, carry):
            b = lax.rem(j, NB)
            pltpu.make_async_copy(g_hbm.at[isrc_v.at[j]], rows_v.at[b],
                                  sems.at[b]).wait()
            pltpu.sync_copy(rows_v.at[b], acc_sh.at[idst_v.at[j]], add=True)
            nxt = j + NB

            @pl.when(nxt < k)
            def _():
                pltpu.async_copy(g_hbm.at[isrc_v.at[nxt]], rows_v.at[b],
                                 sems.at[b])

            return carry

        lax.fori_loop(0, k, step, 0)
        plsc.subcore_barrier()
        pltpu.sync_copy(acc_sh.at[pl.ds(s * rpt, rpt)],
                        out_hbm.at[c, pl.ds(s * rpt, rpt)])

    return scat_kernel(g, srcp, dstp, zeros32)


def _dinv_from(d_ref):
    deg = d_ref[0, :, 0] + d_ref[1, :, 0] + 1.0
    return lax.rsqrt(deg)


def _tc_in(xp, W1, degacc, nt, r):
    def body(x_ref, w_ref, d_ref, o_ref):
        dinv = _dinv_from(d_ref)
        h = jnp.dot(x_ref[...], w_ref[...], preferred_element_type=jnp.float32)
        o_ref[...] = h * dinv[:, None]

    return pl.pallas_call(
        body,
        grid=(nt // r,),
        in_specs=[
            pl.BlockSpec((r, xp.shape[1]), lambda i: (i, 0)),
            pl.BlockSpec(W1.shape, lambda i: (0, 0)),
            pl.BlockSpec((NC, r, 16), lambda i: (0, i, 0)),
        ],
        out_specs=pl.BlockSpec((r, DH), lambda i: (i, 0)),
        out_shape=jax.ShapeDtypeStruct((nt, DH), jnp.float32),
    )(xp, W1, degacc)


def _tc_mid(sacc, g1, degacc, b1, W2, nt, r):
    def body(s_ref, g_ref, d_ref, b_ref, w_ref, o_ref):
        dinv = _dinv_from(d_ref)
        stot = s_ref[0] + s_ref[1] + g_ref[...]
        z = jnp.maximum(stot * dinv[:, None] + b_ref[...], 0.0)
        h = jnp.dot(z, w_ref[...], preferred_element_type=jnp.float32)
        o_ref[...] = h * dinv[:, None]

    return pl.pallas_call(
        body,
        grid=(nt // r,),
        in_specs=[
            pl.BlockSpec((NC, r, DH), lambda i: (0, i, 0)),
            pl.BlockSpec((r, DH), lambda i: (i, 0)),
            pl.BlockSpec((NC, r, 16), lambda i: (0, i, 0)),
            pl.BlockSpec(b1.shape, lambda i: (0, 0)),
            pl.BlockSpec(W2.shape, lambda i: (0, 0)),
        ],
        out_specs=pl.BlockSpec((r, DH), lambda i: (i, 0)),
        out_shape=jax.ShapeDtypeStruct((nt, DH), jnp.float32),
    )(sacc, g1, degacc, b1, W2)


def _tc_head(sacc, g2, degacc, b2, Wfc, bfc, nt, r):
    dout = Wfc.shape[1]

    def body(s_ref, g_ref, d_ref, b_ref, w_ref, bf_ref, o_ref):
        dinv = _dinv_from(d_ref)
        stot = s_ref[0] + s_ref[1] + g_ref[...]
        z = jnp.maximum(stot * dinv[:, None] + b_ref[...], 0.0)
        h = jnp.dot(z, w_ref[...], preferred_element_type=jnp.float32)
        o_ref[...] = jax.nn.sigmoid(h + bf_ref[...])

    return pl.pallas_call(
        body,
        grid=(nt // r,),
        in_specs=[
            pl.BlockSpec((NC, r, DH), lambda i: (0, i, 0)),
            pl.BlockSpec((r, DH), lambda i: (i, 0)),
            pl.BlockSpec((NC, r, 16), lambda i: (0, i, 0)),
            pl.BlockSpec(b2.shape, lambda i: (0, 0)),
            pl.BlockSpec(Wfc.shape, lambda i: (0, 0)),
            pl.BlockSpec(bfc.shape, lambda i: (0, 0)),
        ],
        out_specs=pl.BlockSpec((r, dout), lambda i: (i, 0)),
        out_shape=jax.ShapeDtypeStruct((nt, dout), jnp.float32),
    )(sacc, g2, degacc, b2, Wfc, bfc)


def kernel(x, edge_index, W1, b1, W2, b2, Wfc, bfc):
    n, din = x.shape
    e = edge_index.shape[1]
    r = 1024
    nt = -(-(n + 1) // r) * r
    ept = -(-e // (NW * CH)) * CH
    k = ept // CH
    pad_e = NW * ept - e

    fill = (n + jnp.arange(pad_e, dtype=edge_index.dtype) % (nt - n))
    srcp = jnp.concatenate([edge_index[0], fill]).reshape(NW, k, CH)
    dstp = jnp.concatenate([edge_index[1], fill]).reshape(NW, k, CH)
    xp = jnp.pad(x, ((0, nt - n), (0, 0)))

    zeros16 = jnp.zeros((nt // NS, 16), jnp.float32)
    zeros32 = jnp.zeros((nt // NS, DH), jnp.float32)
    ones16 = jnp.ones((CH, 16), jnp.float32)

    degacc = _sc_degree(dstp, zeros16, ones16, nt)
    g1 = _tc_in(xp, W1, degacc, nt, r)
    s1 = _sc_scatter(g1, srcp, dstp, zeros32, nt)
    g2 = _tc_mid(s1, g1, degacc, b1.reshape(1, DH), W2, nt, r)
    s2 = _sc_scatter(g2, srcp, dstp, zeros32, nt)

    dpad = 8 - Wfc.shape[1]
    wfc_p = jnp.pad(Wfc, ((0, 0), (0, dpad)))
    bfc_p = jnp.pad(bfc, (0, dpad)).reshape(1, 8)
    outp = _tc_head(s2, g2, degacc, b2.reshape(1, DH), wfc_p, bfc_p, nt, r)
    return outp[:n, :Wfc.shape[1]]


# speedup vs baseline: 52.1602x; 1.0767x over previous
import functools

import jax
import jax.numpy as jnp
from jax import lax
from jax.experimental import pallas as pl
from jax.experimental.pallas import tpu as pltpu
from jax.experimental.pallas import tpu_sc as plsc

NC = 2
NS = 16
NW = NC * NS
CH = 128
NB = 4
DH = 32


def _load_index_chunks(ei3_hbm, which, idx_v, wid, k, kw, isem):
    def fire(j, carry):
        cid = wid + j * NW

        @pl.when(j < kw)
        def _():
            pltpu.async_copy(ei3_hbm.at[which, cid], idx_v.at[j], isem)

        return carry

    lax.fori_loop(0, k, fire, 0)

    def drain(j, carry):
        cid = wid + j * NW

        @pl.when(j < kw)
        def _():
            pltpu.make_async_copy(ei3_hbm.at[which, cid], idx_v.at[j], isem).wait()

        return carry

    lax.fori_loop(0, k, drain, 0)


def _sc_degree(ei3, zeros16, ones16, n, k):
    nchunks = ei3.shape[1]
    rpt = n // NS
    mesh = plsc.VectorSubcoreMesh(core_axis_name="c", subcore_axis_name="s")

    @functools.partial(
        pl.kernel,
        out_type=jax.ShapeDtypeStruct((NC, n, 16), jnp.float32),
        mesh=mesh,
        scratch_types=[
            pltpu.VMEM((k, CH), jnp.int32),
            pltpu.VMEM((CH, 16), jnp.float32),
            pltpu.VMEM_SHARED((n, 16), jnp.float32),
            pltpu.SemaphoreType.DMA,
        ],
        compiler_params=pltpu.CompilerParams(use_tc_tiling_on_sc=False),
    )
    def deg_kernel(ei3_hbm, z_hbm, ones_hbm, out_hbm, idx_v, ones_v, acc_sh, isem):
        c = lax.axis_index("c")
        s = lax.axis_index("s")
        wid = c * NS + s
        kw = (nchunks - wid + NW - 1) // NW
        pltpu.sync_copy(ones_hbm, ones_v)
        pltpu.sync_copy(z_hbm, acc_sh.at[pl.ds(s * rpt, rpt)])
        _load_index_chunks(ei3_hbm, 1, idx_v, wid, k, kw, isem)
        plsc.subcore_barrier()

        def step(j, carry):
            @pl.when(j < kw)
            def _():
                pltpu.sync_copy(ones_v, acc_sh.at[idx_v.at[j]], add=True)

            return carry

        lax.fori_loop(0, k, step, 0)
        plsc.subcore_barrier()
        pltpu.sync_copy(acc_sh.at[pl.ds(s * rpt, rpt)],
                        out_hbm.at[c, pl.ds(s * rpt, rpt)])

    return deg_kernel(ei3, zeros16, ones16)


def _sc_scatter(g, ei3, zeros32, n, k):
    nchunks = ei3.shape[1]
    rpt = n // NS
    mesh = plsc.VectorSubcoreMesh(core_axis_name="c", subcore_axis_name="s")

    @functools.partial(
        pl.kernel,
        out_type=jax.ShapeDtypeStruct((NC, n, DH), jnp.float32),
        mesh=mesh,
        scratch_types=[
            pltpu.VMEM((k, CH), jnp.int32),
            pltpu.VMEM((k, CH), jnp.int32),
            pltpu.VMEM((NB, CH, DH), jnp.float32),
            pltpu.VMEM_SHARED((n, DH), jnp.float32),
            pltpu.SemaphoreType.DMA((NB,)),
            pltpu.SemaphoreType.DMA,
        ],
        compiler_params=pltpu.CompilerParams(use_tc_tiling_on_sc=False),
    )
    def scat_kernel(g_hbm, ei3_hbm, z_hbm, out_hbm,
                    isrc_v, idst_v, rows_v, acc_sh, sems, isem):
        c = lax.axis_index("c")
        s = lax.axis_index("s")
        wid = c * NS + s
        kw = (nchunks - wid + NW - 1) // NW
        pltpu.sync_copy(z_hbm, acc_sh.at[pl.ds(s * rpt, rpt)])
        _load_index_chunks(ei3_hbm, 0, isrc_v, wid, k, kw, isem)
        _load_index_chunks(ei3_hbm, 1, idst_v, wid, k, kw, isem)
        plsc.subcore_barrier()

        for b in range(min(NB, k)):
            @pl.when(b < kw)
            def _():
                pltpu.async_copy(g_hbm.at[isrc_v.at[b]], rows_v.at[b], sems.at[b])

        def step(j, carry):
            b = lax.rem(j, NB)

            @pl.when(j < kw)
            def _():
                pltpu.make_async_copy(g_hbm.at[isrc_v.at[j]], rows_v.at[b],
                                      sems.at[b]).wait()
                pltpu.sync_copy(rows_v.at[b], acc_sh.at[idst_v.at[j]], add=True)
                nxt = j + NB

                @pl.when(nxt < kw)
                def _():
                    pltpu.async_copy(g_hbm.at[isrc_v.at[nxt]], rows_v.at[b],
                                     sems.at[b])

            return carry

        lax.fori_loop(0, k, step, 0)
        plsc.subcore_barrier()
        pltpu.sync_copy(acc_sh.at[pl.ds(s * rpt, rpt)],
                        out_hbm.at[c, pl.ds(s * rpt, rpt)])

    return scat_kernel(g, ei3, zeros32)


def _dinv_from(d_ref):
    deg = d_ref[0, :, 0] + d_ref[1, :, 0] + 1.0
    return lax.rsqrt(deg)


def _tc_in(x, W1, degacc, r):
    n = x.shape[0]

    def body(x_ref, w_ref, d_ref, o_ref):
        dinv = _dinv_from(d_ref)
        h = jnp.dot(x_ref[...], w_ref[...], preferred_element_type=jnp.float32)
        o_ref[...] = h * dinv[:, None]

    return pl.pallas_call(
        body,
        grid=(n // r,),
        in_specs=[
            pl.BlockSpec((r, x.shape[1]), lambda i: (i, 0)),
            pl.BlockSpec(W1.shape, lambda i: (0, 0)),
            pl.BlockSpec((NC, r, 16), lambda i: (0, i, 0)),
        ],
        out_specs=pl.BlockSpec((r, DH), lambda i: (i, 0)),
        out_shape=jax.ShapeDtypeStruct((n, DH), jnp.float32),
    )(x, W1, degacc)


def _tc_mid(sacc, g1, degacc, b1, W2, r):
    n = g1.shape[0]

    def body(s_ref, g_ref, d_ref, b_ref, w_ref, o_ref):
        dinv = _dinv_from(d_ref)
        stot = s_ref[0] + s_ref[1] + g_ref[...]
        z = jnp.maximum(stot * dinv[:, None] + b_ref[...], 0.0)
        h = jnp.dot(z, w_ref[...], preferred_element_type=jnp.float32)
        o_ref[...] = h * dinv[:, None]

    return pl.pallas_call(
        body,
        grid=(n // r,),
        in_specs=[
            pl.BlockSpec((NC, r, DH), lambda i: (0, i, 0)),
            pl.BlockSpec((r, DH), lambda i: (i, 0)),
            pl.BlockSpec((NC, r, 16), lambda i: (0, i, 0)),
            pl.BlockSpec(b1.shape, lambda i: (0, 0)),
            pl.BlockSpec(W2.shape, lambda i: (0, 0)),
        ],
        out_specs=pl.BlockSpec((r, DH), lambda i: (i, 0)),
        out_shape=jax.ShapeDtypeStruct((n, DH), jnp.float32),
    )(sacc, g1, degacc, b1, W2)


def _tc_head(sacc, g2, degacc, b2, Wfc, bfc, r):
    n = g2.shape[0]
    dout = Wfc.shape[1]

    def body(s_ref, g_ref, d_ref, b_ref, w_ref, bf_ref, o_ref):
        dinv = _dinv_from(d_ref)
        stot = s_ref[0] + s_ref[1] + g_ref[...]
        z = jnp.maximum(stot * dinv[:, None] + b_ref[...], 0.0)
        h = jnp.dot(z, w_ref[...], preferred_element_type=jnp.float32)
        o_ref[...] = jax.nn.sigmoid(h + bf_ref[...])

    return pl.pallas_call(
        body,
        grid=(n // r,),
        in_specs=[
            pl.BlockSpec((NC, r, DH), lambda i: (0, i, 0)),
            pl.BlockSpec((r, DH), lambda i: (i, 0)),
            pl.BlockSpec((NC, r, 16), lambda i: (0, i, 0)),
            pl.BlockSpec(b2.shape, lambda i: (0, 0)),
            pl.BlockSpec(Wfc.shape, lambda i: (0, 0)),
            pl.BlockSpec(bfc.shape, lambda i: (0, 0)),
        ],
        out_specs=pl.BlockSpec((r, dout), lambda i: (i, 0)),
        out_shape=jax.ShapeDtypeStruct((n, dout), jnp.float32),
    )(sacc, g2, degacc, b2, Wfc, bfc)


def kernel(x, edge_index, W1, b1, W2, b2, Wfc, bfc):
    n, din = x.shape
    e = edge_index.shape[1]
    r = 1000
    assert e % CH == 0 and n % NS == 0 and n % r == 0

    ei3 = edge_index.reshape(2, e // CH, CH)
    k = -(-(e // CH) // NW)

    zeros16 = jnp.zeros((n // NS, 16), jnp.float32)
    zeros32 = jnp.zeros((n // NS, DH), jnp.float32)
    ones16 = jnp.ones((CH, 16), jnp.float32)

    degacc = _sc_degree(ei3, zeros16, ones16, n, k)
    g1 = _tc_in(x, W1, degacc, r)
    s1 = _sc_scatter(g1, ei3, zeros32, n, k)
    g2 = _tc_mid(s1, g1, degacc, b1.reshape(1, DH), W2, r)
    s2 = _sc_scatter(g2, ei3, zeros32, n, k)

    dpad = 8 - Wfc.shape[1]
    wfc_p = jnp.pad(Wfc, ((0, 0), (0, dpad)))
    bfc_p = jnp.pad(bfc, (0, dpad)).reshape(1, 8)
    outp = _tc_head(s2, g2, degacc, b2.reshape(1, DH), wfc_p, bfc_p, r)
    return outp[:, :Wfc.shape[1]]


# speedup vs baseline: 66.3077x; 1.2712x over previous
import functools

import jax
import jax.numpy as jnp
from jax import lax
from jax.experimental import pallas as pl
from jax.experimental.pallas import tpu as pltpu
from jax.experimental.pallas import tpu_sc as plsc

NC = 2
NS = 16
NW = NC * NS
CH = 128
NB = 4
DH = 32


def _load_index_chunks(ei3_hbm, which, idx_v, wid, k, kw, isem):
    def fire(j, carry):
        cid = wid + j * NW

        @pl.when(j < kw)
        def _():
            pltpu.async_copy(ei3_hbm.at[which, cid], idx_v.at[j], isem)

        return carry

    lax.fori_loop(0, k, fire, 0)

    def drain(j, carry):
        cid = wid + j * NW

        @pl.when(j < kw)
        def _():
            pltpu.make_async_copy(ei3_hbm.at[which, cid], idx_v.at[j], isem).wait()

        return carry

    lax.fori_loop(0, k, drain, 0)


def _sc_degree(ei3, zeros32, ones32, nt, k):
    nchunks = ei3.shape[1]
    rpt = nt // NS
    mesh = plsc.VectorSubcoreMesh(core_axis_name="c", subcore_axis_name="s")

    @functools.partial(
        pl.kernel,
        out_type=jax.ShapeDtypeStruct((NC, nt, DH), jnp.float32),
        mesh=mesh,
        scratch_types=[
            pltpu.VMEM((k, CH), jnp.int32),
            pltpu.VMEM((CH, DH), jnp.float32),
            pltpu.VMEM_SHARED((nt, DH), jnp.float32),
            pltpu.SemaphoreType.DMA,
        ],
        compiler_params=pltpu.CompilerParams(use_tc_tiling_on_sc=False),
    )
    def deg_kernel(ei3_hbm, z_hbm, ones_hbm, out_hbm, idx_v, ones_v, acc_sh, isem):
        c = lax.axis_index("c")
        s = lax.axis_index("s")
        wid = c * NS + s
        kw = (nchunks - wid + NW - 1) // NW
        pltpu.sync_copy(ones_hbm, ones_v)
        pltpu.sync_copy(z_hbm, acc_sh.at[pl.ds(s * rpt, rpt)])
        _load_index_chunks(ei3_hbm, 1, idx_v, wid, k, kw, isem)
        plsc.subcore_barrier()

        def step(j, carry):
            @pl.when(j < kw)
            def _():
                pltpu.sync_copy(ones_v, acc_sh.at[idx_v.at[j]], add=True)

            return carry

        lax.fori_loop(0, k, step, 0)
        plsc.subcore_barrier()
        pltpu.sync_copy(acc_sh.at[pl.ds(s * rpt, rpt)],
                        out_hbm.at[c, pl.ds(s * rpt, rpt)])

    return deg_kernel(ei3, zeros32, ones32).reshape(NC, nt // 4, CH)


def _sc_scatter(gp, ei3, zeros32, nt, k):
    g = gp.reshape(nt, DH)
    nchunks = ei3.shape[1]
    rpt = nt // NS
    mesh = plsc.VectorSubcoreMesh(core_axis_name="c", subcore_axis_name="s")

    @functools.partial(
        pl.kernel,
        out_type=jax.ShapeDtypeStruct((NC, nt, DH), jnp.float32),
        mesh=mesh,
        scratch_types=[
            pltpu.VMEM((k, CH), jnp.int32),
            pltpu.VMEM((k, CH), jnp.int32),
            pltpu.VMEM((NB, CH, DH), jnp.float32),
            pltpu.VMEM_SHARED((nt, DH), jnp.float32),
            pltpu.SemaphoreType.DMA((NB,)),
            pltpu.SemaphoreType.DMA,
        ],
        compiler_params=pltpu.CompilerParams(use_tc_tiling_on_sc=False),
    )
    def scat_kernel(g_hbm, ei3_hbm, z_hbm, out_hbm,
                    isrc_v, idst_v, rows_v, acc_sh, sems, isem):
        c = lax.axis_index("c")
        s = lax.axis_index("s")
        wid = c * NS + s
        kw = (nchunks - wid + NW - 1) // NW
        pltpu.sync_copy(z_hbm, acc_sh.at[pl.ds(s * rpt, rpt)])
        _load_index_chunks(ei3_hbm, 0, isrc_v, wid, k, kw, isem)
        _load_index_chunks(ei3_hbm, 1, idst_v, wid, k, kw, isem)
        plsc.subcore_barrier()

        for b in range(min(NB, k)):
            @pl.when(b < kw)
            def _():
                pltpu.async_copy(g_hbm.at[isrc_v.at[b]], rows_v.at[b], sems.at[b])

        def step(j, carry):
            b = lax.rem(j, NB)

            @pl.when(j < kw)
            def _():
                pltpu.make_async_copy(g_hbm.at[isrc_v.at[j]], rows_v.at[b],
                                      sems.at[b]).wait()
                pltpu.sync_copy(rows_v.at[b], acc_sh.at[idst_v.at[j]], add=True)
                nxt = j + NB

                @pl.when(nxt < kw)
                def _():
                    pltpu.async_copy(g_hbm.at[isrc_v.at[nxt]], rows_v.at[b],
                                     sems.at[b])

            return carry

        lax.fori_loop(0, k, step, 0)
        plsc.subcore_barrier()
        pltpu.sync_copy(acc_sh.at[pl.ds(s * rpt, rpt)],
                        out_hbm.at[c, pl.ds(s * rpt, rpt)])

    return scat_kernel(g, ei3, zeros32).reshape(NC, nt // 4, CH)


def _tc_in(x4, W1s, degp, nt, r):
    rp = r // 4

    def body(x_ref, w_ref, d_ref, o_ref):
        dinv = lax.rsqrt(d_ref[0] + d_ref[1] + 1.0)
        h = jnp.dot(x_ref[...], w_ref[...], preferred_element_type=jnp.float32)
        o_ref[...] = h * dinv

    return pl.pallas_call(
        body,
        grid=(nt // r,),
        in_specs=[
            pl.BlockSpec((rp, x4.shape[1]), lambda i: (i, 0)),
            pl.BlockSpec(W1s.shape, lambda i: (0, 0)),
            pl.BlockSpec((NC, rp, CH), lambda i: (0, i, 0)),
        ],
        out_specs=pl.BlockSpec((rp, CH), lambda i: (i, 0)),
        out_shape=jax.ShapeDtypeStruct((nt // 4, CH), jnp.float32),
    )(x4, W1s, degp)


def _tc_mid(sp, gp, degp, b1p, W2s, nt, r):
    rp = r // 4

    def body(s_ref, g_ref, d_ref, b_ref, w_ref, o_ref):
        dinv = lax.rsqrt(d_ref[0] + d_ref[1] + 1.0)
        stot = s_ref[0] + s_ref[1] + g_ref[...]
        z = jnp.maximum(stot * dinv + b_ref[...], 0.0)
        h = jnp.dot(z, w_ref[...], preferred_element_type=jnp.float32)
        o_ref[...] = h * dinv

    return pl.pallas_call(
        body,
        grid=(nt // r,),
        in_specs=[
            pl.BlockSpec((NC, rp, CH), lambda i: (0, i, 0)),
            pl.BlockSpec((rp, CH), lambda i: (i, 0)),
            pl.BlockSpec((NC, rp, CH), lambda i: (0, i, 0)),
            pl.BlockSpec(b1p.shape, lambda i: (0, 0)),
            pl.BlockSpec(W2s.shape, lambda i: (0, 0)),
        ],
        out_specs=pl.BlockSpec((rp, CH), lambda i: (i, 0)),
        out_shape=jax.ShapeDtypeStruct((nt // 4, CH), jnp.float32),
    )(sp, gp, degp, b1p, W2s)


def _tc_head(sp, gp, degp, b2p, Wfs, bfp, nt, r):
    rp = r // 4

    def body(s_ref, g_ref, d_ref, b_ref, w_ref, bf_ref, o_ref):
        dinv = lax.rsqrt(d_ref[0] + d_ref[1] + 1.0)
        stot = s_ref[0] + s_ref[1] + g_ref[...]
        z = jnp.maximum(stot * dinv + b_ref[...], 0.0)
        h = jnp.dot(z, w_ref[...], preferred_element_type=jnp.float32)
        o_ref[...] = jax.nn.sigmoid(h + bf_ref[...])

    return pl.pallas_call(
        body,
        grid=(nt // r,),
        in_specs=[
            pl.BlockSpec((NC, rp, CH), lambda i: (0, i, 0)),
            pl.BlockSpec((rp, CH), lambda i: (i, 0)),
            pl.BlockSpec((NC, rp, CH), lambda i: (0, i, 0)),
            pl.BlockSpec(b2p.shape, lambda i: (0, 0)),
            pl.BlockSpec(Wfs.shape, lambda i: (0, 0)),
            pl.BlockSpec(bfp.shape, lambda i: (0, 0)),
        ],
        out_specs=pl.BlockSpec((rp, Wfs.shape[1]), lambda i: (i, 0)),
        out_shape=jax.ShapeDtypeStruct((nt // 4, Wfs.shape[1]), jnp.float32),
    )(sp, gp, degp, b2p, Wfs, bfp)


def kernel(x, edge_index, W1, b1, W2, b2, Wfc, bfc):
    n, din = x.shape
    e = edge_index.shape[1]
    r = 2048
    nt = -(-n // r) * r
    dout = Wfc.shape[1]
    assert e % CH == 0 and nt % (8 * NS) == 0

    ei3 = edge_index.reshape(2, e // CH, CH)
    k = -(-(e // CH) // NW)

    x4 = jnp.pad(x, ((0, nt - n), (0, 0))).reshape(nt // 4, 4 * din)
    zeros32 = jnp.zeros((nt // NS, DH), jnp.float32)
    ones32 = jnp.ones((CH, DH), jnp.float32)

    eye4 = jnp.eye(4, dtype=jnp.float32)
    W1s = jnp.kron(eye4, W1)
    W2s = jnp.kron(eye4, W2)
    wfc_p = jnp.pad(Wfc, ((0, 0), (0, 8 - dout)))
    Wfs = jnp.kron(eye4, wfc_p)
    b1p = jnp.tile(b1, 4).reshape(1, CH)
    b2p = jnp.tile(b2, 4).reshape(1, CH)
    bfp = jnp.tile(jnp.pad(bfc, (0, 8 - dout)), 4).reshape(1, DH)

    degp = _sc_degree(ei3, zeros32, ones32, nt, k)
    g1p = _tc_in(x4, W1s, degp, nt, r)
    s1p = _sc_scatter(g1p, ei3, zeros32, nt, k)
    g2p = _tc_mid(s1p, g1p, degp, b1p, W2s, nt, r)
    s2p = _sc_scatter(g2p, ei3, zeros32, nt, k)
    outp = _tc_head(s2p, g2p, degp, b2p, Wfs, bfp, nt, r)

    return outp[:n // 4].reshape(n, 8)[:, :dout]
